# baseline (speedup 1.0000x reference)
import numpy as np

import jax
import jax.numpy as jnp
from jax import lax
from jax.experimental import pallas as pl
from jax.experimental.pallas import tpu as pltpu
from jax.experimental.pallas import tpu_sc as plsc

N = 50000
E = 1600000
NPAD = 50048
EPAD = 1638400
ECH = EPAD // 128
NTILES = 32
CPT = ECH // NTILES
BCH = 40
NBUF = 4
ROWS_PER_TILE = NPAD // 16
K1 = 40000
K2 = 32000
INT_MIN = np.int32(-(2 ** 31))


def _zero_fill_buf(zbuf, d):
    z16 = jnp.zeros((16,), jnp.float32)

    def fill_row(r, carry):
        for c2 in range(d // 16):
            zbuf[r, pl.ds(c2 * 16, 16)] = z16
        return carry

    lax.fori_loop(0, 128, fill_row, 0)


def _zero_acc_rows(zbuf, acc, row0):
    def zr(j, carry):
        pltpu.sync_copy(zbuf, acc.at[pl.ds(row0 + j * 128, 128)])
        return carry

    lax.fori_loop(0, ROWS_PER_TILE // 128, zr, 0)
    rem = ROWS_PER_TILE % 128
    if rem:
        pltpu.sync_copy(zbuf.at[pl.ds(0, rem)],
                        acc.at[pl.ds(row0 + (ROWS_PER_TILE - rem), rem)])


def _edge_pass(tbl_hbm, src_hbm, dst_hbm, wid,
               sidx, didx, bufs, sems_g, sems_s, acc):
    nb = NBUF

    def wait_g(b, j):
        pltpu.make_async_copy(tbl_hbm.at[sidx.at[j]], bufs[b], sems_g[b]).wait()

    def wait_s(b, j):
        pltpu.make_async_copy(bufs[b], acc.at[didx.at[j]], sems_s[b]).wait()

    def block(blk, carry):
        base = wid * CPT + blk * BCH
        pltpu.sync_copy(src_hbm.at[pl.ds(base, BCH)], sidx)
        pltpu.sync_copy(dst_hbm.at[pl.ds(base, BCH)], didx)
        for b in range(nb):
            pltpu.async_copy(tbl_hbm.at[sidx.at[b]], bufs[b], sems_g[b])

        def step(q, c2):
            j = q * nb
            for b in range(nb):
                wait_g(b, j + b)
                pltpu.async_copy(bufs[b], acc.at[didx.at[j + b]],
                                 sems_s[b], add=True)
                if b >= 2:
                    b2 = b - 2
                    wait_s(b2, j + b2)
                    pltpu.async_copy(tbl_hbm.at[sidx.at[j + b2 + nb]],
                                     bufs[b2], sems_g[b2])
            for b in range(nb - 2, nb):
                wait_s(b, j + b)
                pltpu.async_copy(tbl_hbm.at[sidx.at[j + b + nb]],
                                 bufs[b], sems_g[b])
            return c2

        lax.fori_loop(0, BCH // nb - 1, step, 0)
        j0 = BCH - nb
        for b in range(nb):
            wait_g(b, j0 + b)
            pltpu.async_copy(bufs[b], acc.at[didx.at[j0 + b]],
                             sems_s[b], add=True)
        for b in range(nb):
            wait_s(b, j0 + b)
        return carry

    lax.fori_loop(0, CPT // BCH, block, 0)


def _make_agg_kernel(d, n_tables):
    mesh = plsc.VectorSubcoreMesh(core_axis_name="c", subcore_axis_name="s")
    out_type = jax.ShapeDtypeStruct((n_tables * 2 * NPAD, d), jnp.float32)
    scratch = (
        [pltpu.VMEM((BCH, 128), jnp.int32),
         pltpu.VMEM((BCH, 128), jnp.int32)]
        + [pltpu.VMEM((128, d), jnp.float32) for _ in range(NBUF)]
        + [pltpu.VMEM_SHARED((NPAD, d), jnp.float32)]
        + [pltpu.SemaphoreType.DMA for _ in range(2 * NBUF)]
    )

    def body(*refs):
        tbls = refs[:n_tables]
        src_hbm, dst_hbm, out_hbm = refs[n_tables:n_tables + 3]
        rest = refs[n_tables + 3:]
        sidx, didx = rest[0], rest[1]
        bufs = rest[2:2 + NBUF]
        acc = rest[2 + NBUF]
        sems_g = rest[3 + NBUF:3 + 2 * NBUF]
        sems_s = rest[3 + 2 * NBUF:3 + 3 * NBUF]
        c = lax.axis_index("c")
        s = lax.axis_index("s")
        wid = c * 16 + s
        row0 = s * ROWS_PER_TILE

        for p in range(n_tables):
            _zero_fill_buf(bufs[0], d)
            _zero_acc_rows(bufs[0], acc, row0)
            plsc.subcore_barrier()
            _edge_pass(tbls[p], src_hbm, dst_hbm, wid,
                       sidx, didx, bufs, sems_g, sems_s, acc)
            plsc.subcore_barrier()
            out0 = (p * 2 + c) * NPAD + row0
            pltpu.sync_copy(acc.at[pl.ds(row0, ROWS_PER_TILE)],
                            out_hbm.at[pl.ds(out0, ROWS_PER_TILE)])
            if p + 1 < n_tables:
                plsc.subcore_barrier()

    return pl.kernel(
        body, out_type=out_type, mesh=mesh, scratch_types=scratch,
        compiler_params=pltpu.CompilerParams(use_tc_tiling_on_sc=False))


_agg1 = _make_agg_kernel(16, 1)
_agg2 = _make_agg_kernel(32, 2)


NB = 8
RB = NPAD // NB


def _sortable_key(score):
    bits = lax.bitcast_convert_type(score, jnp.int32)
    return jnp.where(bits >= 0, bits, INT_MIN - bits)


def _conv1_dense_body(agg_a_ref, agg_b_ref, xp_ref, wrel_ref, brel_ref,
                      wroot_ref, p1_ref, h_ref, score_ref):
    agg = agg_a_ref[...] + agg_b_ref[...]
    h = jnp.dot(agg, wrel_ref[...], preferred_element_type=jnp.float32)
    h += jnp.dot(xp_ref[...], wroot_ref[...], preferred_element_type=jnp.float32)
    h = jnp.maximum(h + brel_ref[...], 0.0)
    h_ref[...] = h
    p1 = p1_ref[...]
    pnorm = jnp.sqrt(jnp.sum(p1 * p1))
    sc = lax.dot_general(h, p1, (((1,), (1,)), ((), ())),
                         preferred_element_type=jnp.float32)
    score_ref[...] = jnp.tanh(sc / pnorm)


_conv1_dense = pl.pallas_call(
    _conv1_dense_body,
    grid=(NB,),
    in_specs=[
        pl.BlockSpec((RB, 16), lambda i: (i, 0)),
        pl.BlockSpec((RB, 16), lambda i: (NB + i, 0)),
        pl.BlockSpec((RB, 16), lambda i: (i, 0)),
        pl.BlockSpec((16, 64), lambda i: (0, 0)),
        pl.BlockSpec((1, 64), lambda i: (0, 0)),
        pl.BlockSpec((16, 64), lambda i: (0, 0)),
        pl.BlockSpec((1, 64), lambda i: (0, 0)),
    ],
    out_specs=(
        pl.BlockSpec((RB, 64), lambda i: (i, 0)),
        pl.BlockSpec((RB, 1), lambda i: (i, 0)),
    ),
    out_shape=(
        jax.ShapeDtypeStruct((NPAD, 64), jnp.float32),
        jax.ShapeDtypeStruct((NPAD, 1), jnp.float32),
    ),
)


def _thr_body(k, score_ref, mask_ref, thr_ref):
    score = score_ref[...]
    if mask_ref is None:
        idx = (lax.broadcasted_iota(jnp.int32, score.shape, 0) * 128
               + lax.broadcasted_iota(jnp.int32, score.shape, 1))
        valid = idx < N
    else:
        valid = mask_ref[...] > 0.5
    skey = jnp.where(valid, _sortable_key(score), INT_MIN)
    kf = jnp.float32(k)

    def count_ge(t):
        return jnp.sum(jnp.where(skey >= t, 1.0, 0.0))

    t0 = jnp.where(count_ge(jnp.int32(0)) >= kf, jnp.int32(0), INT_MIN)

    def step(i, t):
        cand = t + lax.shift_left(jnp.int32(1), jnp.int32(30) - i)
        return jnp.where(count_ge(cand) >= kf, cand, t)

    thr_ref[...] = jnp.full((1, 1), 0, jnp.int32) + lax.fori_loop(0, 31, step, t0)


_thr1 = pl.pallas_call(
    lambda score_ref, thr_ref: _thr_body(K1, score_ref, None, thr_ref),
    out_shape=jax.ShapeDtypeStruct((1, 1), jnp.int32),
)

_thr2 = pl.pallas_call(
    lambda score_ref, mask_ref, thr_ref: _thr_body(K2, score_ref, mask_ref, thr_ref),
    out_shape=jax.ShapeDtypeStruct((1, 1), jnp.int32),
)


def _pool1_body(h_ref, score_ref, thr_ref, g0_ref, g1_ref, alive_ref, x1p_ref):
    i = pl.program_id(0)
    h = h_ref[...]
    score = score_ref[...]
    rows = i * RB + lax.broadcasted_iota(jnp.int32, (RB, 1), 0)
    valid = rows < N
    skey = jnp.where(valid, _sortable_key(score), INT_MIN)
    alive = skey >= thr_ref[0, 0]
    alive_f = jnp.where(alive, 1.0, 0.0)
    g = h * score * alive_f
    g0_ref[...] = g[:, 0:32]
    g1_ref[...] = g[:, 32:64]
    alive_ref[...] = alive_f
    prod = h * score
    bmax = jnp.max(jnp.where(alive, prod, -1e30), axis=0, keepdims=True)
    bsum = jnp.sum(g, axis=0, keepdims=True)
    blk = jnp.concatenate([bmax, bsum], axis=1)

    @pl.when(i == 0)
    def _():
        x1p_ref[...] = blk

    @pl.when(i > 0)
    def _():
        prev = x1p_ref[...]
        x1p_ref[...] = jnp.concatenate(
            [jnp.maximum(prev[:, 0:64], blk[:, 0:64]),
             prev[:, 64:128] + blk[:, 64:128]], axis=1)


_pool1 = pl.pallas_call(
    _pool1_body,
    grid=(NB,),
    in_specs=[
        pl.BlockSpec((RB, 64), lambda i: (i, 0)),
        pl.BlockSpec((RB, 1), lambda i: (i, 0)),
        pl.BlockSpec((1, 1), lambda i: (0, 0)),
    ],
    out_specs=(
        pl.BlockSpec((RB, 32), lambda i: (i, 0)),
        pl.BlockSpec((RB, 32), lambda i: (i, 0)),
        pl.BlockSpec((RB, 1), lambda i: (i, 0)),
        pl.BlockSpec((1, 128), lambda i: (0, 0)),
    ),
    out_shape=(
        jax.ShapeDtypeStruct((NPAD, 32), jnp.float32),
        jax.ShapeDtypeStruct((NPAD, 32), jnp.float32),
        jax.ShapeDtypeStruct((NPAD, 1), jnp.float32),
        jax.ShapeDtypeStruct((1, 128), jnp.float32),
    ),
)


def _conv2_dense_body(agg_a_ref, agg_b_ref, agg_c_ref, agg_d_ref,
                      g0_ref, g1_ref, wrel_ref, brel_ref, wroot_ref, p2_ref,
                      h_ref, score_ref):
    agg_lo = agg_a_ref[...] + agg_b_ref[...]
    agg_hi = agg_c_ref[...] + agg_d_ref[...]
    h = jnp.dot(agg_lo, wrel_ref[0:32, :], preferred_element_type=jnp.float32)
    h += jnp.dot(agg_hi, wrel_ref[32:64, :], preferred_element_type=jnp.float32)
    h += jnp.dot(g0_ref[...], wroot_ref[0:32, :], preferred_element_type=jnp.float32)
    h += jnp.dot(g1_ref[...], wroot_ref[32:64, :], preferred_element_type=jnp.float32)
    h = jnp.maximum(h + brel_ref[...], 0.0)
    h_ref[...] = h
    p2 = p2_ref[...]
    pnorm = jnp.sqrt(jnp.sum(p2 * p2))
    sc = lax.dot_general(h, p2, (((1,), (1,)), ((), ())),
                         preferred_element_type=jnp.float32)
    score_ref[...] = jnp.tanh(sc / pnorm)


_conv2_dense = pl.pallas_call(
    _conv2_dense_body,
    grid=(NB,),
    in_specs=[
        pl.BlockSpec((RB, 32), lambda i: (i, 0)),
        pl.BlockSpec((RB, 32), lambda i: (NB + i, 0)),
        pl.BlockSpec((RB, 32), lambda i: (2 * NB + i, 0)),
        pl.BlockSpec((RB, 32), lambda i: (3 * NB + i, 0)),
        pl.BlockSpec((RB, 32), lambda i: (i, 0)),
        pl.BlockSpec((RB, 32), lambda i: (i, 0)),
        pl.BlockSpec((64, 64), lambda i: (0, 0)),
        pl.BlockSpec((1, 64), lambda i: (0, 0)),
        pl.BlockSpec((64, 64), lambda i: (0, 0)),
        pl.BlockSpec((1, 64), lambda i: (0, 0)),
    ],
    out_specs=(
        pl.BlockSpec((RB, 64), lambda i: (i, 0)),
        pl.BlockSpec((RB, 1), lambda i: (i, 0)),
    ),
    out_shape=(
        jax.ShapeDtypeStruct((NPAD, 64), jnp.float32),
        jax.ShapeDtypeStruct((NPAD, 1), jnp.float32),
    ),
)


def _pool2_body(h_ref, score_ref, alive_ref, thr_ref, x2p_ref):
    i = pl.program_id(0)
    h = h_ref[...]
    score = score_ref[...]
    alive1 = alive_ref[...] > 0.5
    skey = jnp.where(alive1, _sortable_key(score), INT_MIN)
    alive2 = alive1 & (skey >= thr_ref[0, 0])
    prod = h * score
    bmax = jnp.max(jnp.where(alive2, prod, -1e30), axis=0, keepdims=True)
    bsum = jnp.sum(jnp.where(alive2, prod, 0.0), axis=0, keepdims=True)
    blk = jnp.concatenate([bmax, bsum], axis=1)

    @pl.when(i == 0)
    def _():
        x2p_ref[...] = blk

    @pl.when(i > 0)
    def _():
        prev = x2p_ref[...]
        x2p_ref[...] = jnp.concatenate(
            [jnp.maximum(prev[:, 0:64], blk[:, 0:64]),
             prev[:, 64:128] + blk[:, 64:128]], axis=1)


_pool2 = pl.pallas_call(
    _pool2_body,
    grid=(NB,),
    in_specs=[
        pl.BlockSpec((RB, 64), lambda i: (i, 0)),
        pl.BlockSpec((RB, 1), lambda i: (i, 0)),
        pl.BlockSpec((RB, 1), lambda i: (i, 0)),
        pl.BlockSpec((1, 1), lambda i: (0, 0)),
    ],
    out_specs=pl.BlockSpec((1, 128), lambda i: (0, 0)),
    out_shape=jax.ShapeDtypeStruct((1, 128), jnp.float32),
)


def _head_body(x1p_ref, x2p_ref, a1w_ref, a1b_ref, a5w_ref, a5b_ref, out_ref):
    x1p = x1p_ref[...]
    x2p = x2p_ref[...]
    x1 = jnp.concatenate([x1p[:, 0:64], x1p[:, 64:128] / jnp.float32(K1)], axis=1)
    x2 = jnp.concatenate([x2p[:, 0:64], x2p[:, 64:128] / jnp.float32(K2)], axis=1)
    xp = x1 + x2
    a1 = jnp.maximum(
        jnp.dot(xp, a1w_ref[...], preferred_element_type=jnp.float32)
        + a1b_ref[...], 0.0)
    a5 = jnp.dot(a1, a5w_ref[...], preferred_element_type=jnp.float32) \
        + a5b_ref[...]
    out_ref[...] = jax.nn.softmax(a5, axis=0)


_head = pl.pallas_call(
    _head_body,
    out_shape=jax.ShapeDtypeStruct((1, 10), jnp.float32),
)


@jax.jit
def kernel(x, edge_index, batch, W_rel1, b_rel1, W_root1, p1,
           W_rel2, b_rel2, W_root2, p2, A1, bA1, A5, bA5):
    src = edge_index[0]
    dst = edge_index[1]
    pad = jnp.full((EPAD - E,), N, jnp.int32)
    srcc = jnp.concatenate([src, pad]).reshape(ECH, 128)
    dstc = jnp.concatenate([dst, pad]).reshape(ECH, 128)

    x_pad = jnp.zeros((NPAD, 16), jnp.float32).at[:N, :12].set(x)
    wrel1 = jnp.zeros((16, 64), jnp.float32).at[:12, :].set(W_rel1)
    wroot1 = jnp.zeros((16, 64), jnp.float32).at[:12, :].set(W_root1)

    aggp1 = _agg1(x_pad, srcc, dstc)
    h1, score1 = _conv1_dense(
        aggp1, aggp1, x_pad, wrel1, b_rel1.reshape(1, 64), wroot1,
        p1.reshape(1, 64))
    thr1 = _thr1(score1.reshape(NPAD // 128, 128))
    g0, g1, alive, x1p = _pool1(h1, score1, thr1)

    aggp2 = _agg2(g0, g1, srcc, dstc)
    h2, score2 = _conv2_dense(
        aggp2, aggp2, aggp2, aggp2, g0, g1,
        W_rel2, b_rel2.reshape(1, 64), W_root2, p2.reshape(1, 64))
    thr2 = _thr2(score2.reshape(NPAD // 128, 128), alive.reshape(NPAD // 128, 128))
    x2p = _pool2(h2, score2, alive, thr2)

    out = _head(x1p, x2p, A1, bA1.reshape(1, 64), A5, bA5.reshape(1, 10))
    return out[None]


# speedup vs baseline: 1.4734x; 1.4734x over previous
import numpy as np

import jax
import jax.numpy as jnp
from jax import lax
from jax.experimental import pallas as pl
from jax.experimental.pallas import tpu as pltpu
from jax.experimental.pallas import tpu_sc as plsc

N = 50000
E = 1600000
NPAD = 50048
EPAD = 1605632
ECH = EPAD // 128
NTILES = 32
CPT = ECH // NTILES
BCH = 56
NBUF = 2
ROWS_PER_TILE = NPAD // 16
K1 = 40000
K2 = 32000
INT_MIN = np.int32(-(2 ** 31))


def _zero_fill_buf(zbuf, d):
    z16 = jnp.zeros((16,), jnp.float32)

    def fill_row(r, carry):
        for c2 in range(d // 16):
            zbuf[r, pl.ds(c2 * 16, 16)] = z16
        return carry

    lax.fori_loop(0, 128, fill_row, 0)


def _zero_acc_rows(zbuf, acc, row0):
    def zr(j, carry):
        pltpu.sync_copy(zbuf, acc.at[pl.ds(row0 + j * 128, 128)])
        return carry

    lax.fori_loop(0, ROWS_PER_TILE // 128, zr, 0)
    rem = ROWS_PER_TILE % 128
    if rem:
        pltpu.sync_copy(zbuf.at[pl.ds(0, rem)],
                        acc.at[pl.ds(row0 + (ROWS_PER_TILE - rem), rem)])


def _edge_pass(tbl_hbm, src_hbm, dst_hbm, wid,
               sidx, didx, bufs, sems_g, sems_s, acc):
    nb = NBUF

    def wait_g(b, j):
        pltpu.make_async_copy(tbl_hbm.at[sidx.at[j]], bufs[b], sems_g[b]).wait()

    def wait_s(b, j):
        pltpu.make_async_copy(bufs[b], acc.at[didx.at[j]], sems_s[b]).wait()

    def block(blk, carry):
        base = wid * CPT + blk * BCH
        pltpu.sync_copy(src_hbm.at[pl.ds(base, BCH)], sidx)
        pltpu.sync_copy(dst_hbm.at[pl.ds(base, BCH)], didx)
        for b in range(nb):
            pltpu.async_copy(tbl_hbm.at[sidx.at[b]], bufs[b], sems_g[b])

        def step(q, c2):
            j = q * nb
            for b in range(nb):
                wait_g(b, j + b)
                pltpu.async_copy(bufs[b], acc.at[didx.at[j + b]],
                                 sems_s[b], add=True)
                if b >= 2:
                    b2 = b - 2
                    wait_s(b2, j + b2)
                    pltpu.async_copy(tbl_hbm.at[sidx.at[j + b2 + nb]],
                                     bufs[b2], sems_g[b2])
            for b in range(nb - 2, nb):
                wait_s(b, j + b)
                pltpu.async_copy(tbl_hbm.at[sidx.at[j + b + nb]],
                                 bufs[b], sems_g[b])
            return c2

        lax.fori_loop(0, BCH // nb - 1, step, 0)
        j0 = BCH - nb
        for b in range(nb):
            wait_g(b, j0 + b)
            pltpu.async_copy(bufs[b], acc.at[didx.at[j0 + b]],
                             sems_s[b], add=True)
        for b in range(nb):
            wait_s(b, j0 + b)
        return carry

    lax.fori_loop(0, CPT // BCH, block, 0)


def _make_agg_kernel(d, n_tables):
    mesh = plsc.VectorSubcoreMesh(core_axis_name="c", subcore_axis_name="s")
    out_type = jax.ShapeDtypeStruct((n_tables * 2 * NPAD, d), jnp.float32)
    scratch = (
        [pltpu.VMEM((BCH, 128), jnp.int32),
         pltpu.VMEM((BCH, 128), jnp.int32)]
        + [pltpu.VMEM((128, d), jnp.float32) for _ in range(NBUF)]
        + [pltpu.VMEM_SHARED((NPAD, d), jnp.float32)]
        + [pltpu.SemaphoreType.DMA for _ in range(2 * NBUF)]
    )

    def body(*refs):
        tbls = refs[:n_tables]
        src_hbm, dst_hbm, out_hbm = refs[n_tables:n_tables + 3]
        rest = refs[n_tables + 3:]
        sidx, didx = rest[0], rest[1]
        bufs = rest[2:2 + NBUF]
        acc = rest[2 + NBUF]
        sems_g = rest[3 + NBUF:3 + 2 * NBUF]
        sems_s = rest[3 + 2 * NBUF:3 + 3 * NBUF]
        c = lax.axis_index("c")
        s = lax.axis_index("s")
        wid = c * 16 + s
        row0 = s * ROWS_PER_TILE

        for p in range(n_tables):
            _zero_fill_buf(bufs[0], d)
            _zero_acc_rows(bufs[0], acc, row0)
            plsc.subcore_barrier()
            _edge_pass(tbls[p], src_hbm, dst_hbm, wid,
                       sidx, didx, bufs, sems_g, sems_s, acc)
            plsc.subcore_barrier()
            out0 = (p * 2 + c) * NPAD + row0
            pltpu.sync_copy(acc.at[pl.ds(row0, ROWS_PER_TILE)],
                            out_hbm.at[pl.ds(out0, ROWS_PER_TILE)])
            if p + 1 < n_tables:
                plsc.subcore_barrier()

    return pl.kernel(
        body, out_type=out_type, mesh=mesh, scratch_types=scratch,
        compiler_params=pltpu.CompilerParams(use_tc_tiling_on_sc=False))


_agg1 = _make_agg_kernel(16, 1)
_agg2 = _make_agg_kernel(32, 2)


NB = 8
RB = NPAD // NB


def _sortable_key(score):
    bits = lax.bitcast_convert_type(score, jnp.int32)
    return jnp.where(bits >= 0, bits, INT_MIN - bits)


def _conv1_dense_body(agg_a_ref, agg_b_ref, xp_ref, wrel_ref, brel_ref,
                      wroot_ref, p1_ref, h_ref, score_ref):
    agg = agg_a_ref[...] + agg_b_ref[...]
    h = jnp.dot(agg, wrel_ref[...], preferred_element_type=jnp.float32)
    h += jnp.dot(xp_ref[...], wroot_ref[...], preferred_element_type=jnp.float32)
    h = jnp.maximum(h + brel_ref[...], 0.0)
    h_ref[...] = h
    p1 = p1_ref[...]
    pnorm = jnp.sqrt(jnp.sum(p1 * p1))
    sc = lax.dot_general(h, p1, (((1,), (1,)), ((), ())),
                         preferred_element_type=jnp.float32)
    score_ref[...] = jnp.tanh(sc / pnorm)


_conv1_dense = pl.pallas_call(
    _conv1_dense_body,
    grid=(NB,),
    in_specs=[
        pl.BlockSpec((RB, 16), lambda i: (i, 0)),
        pl.BlockSpec((RB, 16), lambda i: (NB + i, 0)),
        pl.BlockSpec((RB, 16), lambda i: (i, 0)),
        pl.BlockSpec((16, 64), lambda i: (0, 0)),
        pl.BlockSpec((1, 64), lambda i: (0, 0)),
        pl.BlockSpec((16, 64), lambda i: (0, 0)),
        pl.BlockSpec((1, 64), lambda i: (0, 0)),
    ],
    out_specs=(
        pl.BlockSpec((RB, 64), lambda i: (i, 0)),
        pl.BlockSpec((RB, 1), lambda i: (i, 0)),
    ),
    out_shape=(
        jax.ShapeDtypeStruct((NPAD, 64), jnp.float32),
        jax.ShapeDtypeStruct((NPAD, 1), jnp.float32),
    ),
)


def _thr_body(k, score_ref, mask_ref, thr_ref):
    score = score_ref[...]
    if mask_ref is None:
        idx = (lax.broadcasted_iota(jnp.int32, score.shape, 0) * 128
               + lax.broadcasted_iota(jnp.int32, score.shape, 1))
        valid = idx < N
    else:
        valid = mask_ref[...] > 0.5
    skey = jnp.where(valid, _sortable_key(score), INT_MIN)
    kf = jnp.float32(k)

    def count_ge(t):
        return jnp.sum(jnp.where(skey >= t, 1.0, 0.0))

    t0 = jnp.where(count_ge(jnp.int32(0)) >= kf, jnp.int32(0), INT_MIN)

    def step(i, t):
        cand = t + lax.shift_left(jnp.int32(1), jnp.int32(30) - i)
        return jnp.where(count_ge(cand) >= kf, cand, t)

    thr_ref[...] = jnp.full((1, 1), 0, jnp.int32) + lax.fori_loop(0, 31, step, t0)


_thr1 = pl.pallas_call(
    lambda score_ref, thr_ref: _thr_body(K1, score_ref, None, thr_ref),
    out_shape=jax.ShapeDtypeStruct((1, 1), jnp.int32),
)

_thr2 = pl.pallas_call(
    lambda score_ref, mask_ref, thr_ref: _thr_body(K2, score_ref, mask_ref, thr_ref),
    out_shape=jax.ShapeDtypeStruct((1, 1), jnp.int32),
)


def _pool1_body(h_ref, score_ref, thr_ref, g0_ref, g1_ref, alive_ref, x1p_ref):
    i = pl.program_id(0)
    h = h_ref[...]
    score = score_ref[...]
    rows = i * RB + lax.broadcasted_iota(jnp.int32, (RB, 1), 0)
    valid = rows < N
    skey = jnp.where(valid, _sortable_key(score), INT_MIN)
    alive = skey >= thr_ref[0, 0]
    alive_f = jnp.where(alive, 1.0, 0.0)
    g = h * score * alive_f
    g0_ref[...] = g[:, 0:32]
    g1_ref[...] = g[:, 32:64]
    alive_ref[...] = alive_f
    prod = h * score
    bmax = jnp.max(jnp.where(alive, prod, -1e30), axis=0, keepdims=True)
    bsum = jnp.sum(g, axis=0, keepdims=True)
    blk = jnp.concatenate([bmax, bsum], axis=1)

    @pl.when(i == 0)
    def _():
        x1p_ref[...] = blk

    @pl.when(i > 0)
    def _():
        prev = x1p_ref[...]
        x1p_ref[...] = jnp.concatenate(
            [jnp.maximum(prev[:, 0:64], blk[:, 0:64]),
             prev[:, 64:128] + blk[:, 64:128]], axis=1)


_pool1 = pl.pallas_call(
    _pool1_body,
    grid=(NB,),
    in_specs=[
        pl.BlockSpec((RB, 64), lambda i: (i, 0)),
        pl.BlockSpec((RB, 1), lambda i: (i, 0)),
        pl.BlockSpec((1, 1), lambda i: (0, 0)),
    ],
    out_specs=(
        pl.BlockSpec((RB, 32), lambda i: (i, 0)),
        pl.BlockSpec((RB, 32), lambda i: (i, 0)),
        pl.BlockSpec((RB, 1), lambda i: (i, 0)),
        pl.BlockSpec((1, 128), lambda i: (0, 0)),
    ),
    out_shape=(
        jax.ShapeDtypeStruct((NPAD, 32), jnp.float32),
        jax.ShapeDtypeStruct((NPAD, 32), jnp.float32),
        jax.ShapeDtypeStruct((NPAD, 1), jnp.float32),
        jax.ShapeDtypeStruct((1, 128), jnp.float32),
    ),
)


def _conv2_dense_body(agg_a_ref, agg_b_ref, agg_c_ref, agg_d_ref,
                      g0_ref, g1_ref, wrel_ref, brel_ref, wroot_ref, p2_ref,
                      h_ref, score_ref):
    agg_lo = agg_a_ref[...] + agg_b_ref[...]
    agg_hi = agg_c_ref[...] + agg_d_ref[...]
    h = jnp.dot(agg_lo, wrel_ref[0:32, :], preferred_element_type=jnp.float32)
    h += jnp.dot(agg_hi, wrel_ref[32:64, :], preferred_element_type=jnp.float32)
    h += jnp.dot(g0_ref[...], wroot_ref[0:32, :], preferred_element_type=jnp.float32)
    h += jnp.dot(g1_ref[...], wroot_ref[32:64, :], preferred_element_type=jnp.float32)
    h = jnp.maximum(h + brel_ref[...], 0.0)
    h_ref[...] = h
    p2 = p2_ref[...]
    pnorm = jnp.sqrt(jnp.sum(p2 * p2))
    sc = lax.dot_general(h, p2, (((1,), (1,)), ((), ())),
                         preferred_element_type=jnp.float32)
    score_ref[...] = jnp.tanh(sc / pnorm)


_conv2_dense = pl.pallas_call(
    _conv2_dense_body,
    grid=(NB,),
    in_specs=[
        pl.BlockSpec((RB, 32), lambda i: (i, 0)),
        pl.BlockSpec((RB, 32), lambda i: (NB + i, 0)),
        pl.BlockSpec((RB, 32), lambda i: (2 * NB + i, 0)),
        pl.BlockSpec((RB, 32), lambda i: (3 * NB + i, 0)),
        pl.BlockSpec((RB, 32), lambda i: (i, 0)),
        pl.BlockSpec((RB, 32), lambda i: (i, 0)),
        pl.BlockSpec((64, 64), lambda i: (0, 0)),
        pl.BlockSpec((1, 64), lambda i: (0, 0)),
        pl.BlockSpec((64, 64), lambda i: (0, 0)),
        pl.BlockSpec((1, 64), lambda i: (0, 0)),
    ],
    out_specs=(
        pl.BlockSpec((RB, 64), lambda i: (i, 0)),
        pl.BlockSpec((RB, 1), lambda i: (i, 0)),
    ),
    out_shape=(
        jax.ShapeDtypeStruct((NPAD, 64), jnp.float32),
        jax.ShapeDtypeStruct((NPAD, 1), jnp.float32),
    ),
)


def _pool2_body(h_ref, score_ref, alive_ref, thr_ref, x2p_ref):
    i = pl.program_id(0)
    h = h_ref[...]
    score = score_ref[...]
    alive1 = alive_ref[...] > 0.5
    skey = jnp.where(alive1, _sortable_key(score), INT_MIN)
    alive2 = alive1 & (skey >= thr_ref[0, 0])
    prod = h * score
    bmax = jnp.max(jnp.where(alive2, prod, -1e30), axis=0, keepdims=True)
    bsum = jnp.sum(jnp.where(alive2, prod, 0.0), axis=0, keepdims=True)
    blk = jnp.concatenate([bmax, bsum], axis=1)

    @pl.when(i == 0)
    def _():
        x2p_ref[...] = blk

    @pl.when(i > 0)
    def _():
        prev = x2p_ref[...]
        x2p_ref[...] = jnp.concatenate(
            [jnp.maximum(prev[:, 0:64], blk[:, 0:64]),
             prev[:, 64:128] + blk[:, 64:128]], axis=1)


_pool2 = pl.pallas_call(
    _pool2_body,
    grid=(NB,),
    in_specs=[
        pl.BlockSpec((RB, 64), lambda i: (i, 0)),
        pl.BlockSpec((RB, 1), lambda i: (i, 0)),
        pl.BlockSpec((RB, 1), lambda i: (i, 0)),
        pl.BlockSpec((1, 1), lambda i: (0, 0)),
    ],
    out_specs=pl.BlockSpec((1, 128), lambda i: (0, 0)),
    out_shape=jax.ShapeDtypeStruct((1, 128), jnp.float32),
)


def _head_body(x1p_ref, x2p_ref, a1w_ref, a1b_ref, a5w_ref, a5b_ref, out_ref):
    x1p = x1p_ref[...]
    x2p = x2p_ref[...]
    x1 = jnp.concatenate([x1p[:, 0:64], x1p[:, 64:128] / jnp.float32(K1)], axis=1)
    x2 = jnp.concatenate([x2p[:, 0:64], x2p[:, 64:128] / jnp.float32(K2)], axis=1)
    xp = x1 + x2
    a1 = jnp.maximum(
        jnp.dot(xp, a1w_ref[...], preferred_element_type=jnp.float32)
        + a1b_ref[...], 0.0)
    a5 = jnp.dot(a1, a5w_ref[...], preferred_element_type=jnp.float32) \
        + a5b_ref[...]
    out_ref[...] = jax.nn.softmax(a5, axis=0)


_head = pl.pallas_call(
    _head_body,
    out_shape=jax.ShapeDtypeStruct((1, 10), jnp.float32),
)


@jax.jit
def kernel(x, edge_index, batch, W_rel1, b_rel1, W_root1, p1,
           W_rel2, b_rel2, W_root2, p2, A1, bA1, A5, bA5):
    src = edge_index[0]
    dst = edge_index[1]
    pad = jnp.full((EPAD - E,), N, jnp.int32)
    srcc = jnp.concatenate([src, pad]).reshape(ECH, 128)
    dstc = jnp.concatenate([dst, pad]).reshape(ECH, 128)

    x_pad = jnp.zeros((NPAD, 16), jnp.float32).at[:N, :12].set(x)
    wrel1 = jnp.zeros((16, 64), jnp.float32).at[:12, :].set(W_rel1)
    wroot1 = jnp.zeros((16, 64), jnp.float32).at[:12, :].set(W_root1)

    aggp1 = _agg1(x_pad, srcc, dstc)
    h1, score1 = _conv1_dense(
        aggp1, aggp1, x_pad, wrel1, b_rel1.reshape(1, 64), wroot1,
        p1.reshape(1, 64))
    thr1 = _thr1(score1.reshape(NPAD // 128, 128))
    g0, g1, alive, x1p = _pool1(h1, score1, thr1)

    aggp2 = _agg2(g0, g1, srcc, dstc)
    h2, score2 = _conv2_dense(
        aggp2, aggp2, aggp2, aggp2, g0, g1,
        W_rel2, b_rel2.reshape(1, 64), W_root2, p2.reshape(1, 64))
    thr2 = _thr2(score2.reshape(NPAD // 128, 128), alive.reshape(NPAD // 128, 128))
    x2p = _pool2(h2, score2, alive, thr2)

    out = _head(x1p, x2p, A1, bA1.reshape(1, 64), A5, bA5.reshape(1, 10))
    return out[None]


# speedup vs baseline: 1.4834x; 1.0068x over previous
import numpy as np

import jax
import jax.numpy as jnp
from jax import lax
from jax.experimental import pallas as pl
from jax.experimental.pallas import tpu as pltpu
from jax.experimental.pallas import tpu_sc as plsc

N = 50000
E = 1600000
NPAD = 50048
EPAD = 1605632
ECH = EPAD // 128
NTILES = 32
CPT = ECH // NTILES
BCH = 56
NBUF = 2
ROWS_PER_TILE = NPAD // 16
K1 = 40000
K2 = 32000
INT_MIN = np.int32(-(2 ** 31))


def _zero_fill_buf(zbuf, d):
    z16 = jnp.zeros((16,), jnp.float32)

    def fill_row(r, carry):
        for c2 in range(d // 16):
            zbuf[r, pl.ds(c2 * 16, 16)] = z16
        return carry

    lax.fori_loop(0, 128, fill_row, 0)


def _zero_acc_rows(zbuf, acc, row0):
    def zr(j, carry):
        pltpu.sync_copy(zbuf, acc.at[pl.ds(row0 + j * 128, 128)])
        return carry

    lax.fori_loop(0, ROWS_PER_TILE // 128, zr, 0)
    rem = ROWS_PER_TILE % 128
    if rem:
        pltpu.sync_copy(zbuf.at[pl.ds(0, rem)],
                        acc.at[pl.ds(row0 + (ROWS_PER_TILE - rem), rem)])


def _edge_pass(tbl_hbm, src_hbm, dst_hbm, wid,
               sidx, didx, bufs, sems_g, sems_s, acc):
    nb = NBUF

    def wait_g(b, j):
        pltpu.make_async_copy(tbl_hbm.at[sidx.at[j]], bufs[b], sems_g[b]).wait()

    def wait_s(b, j):
        pltpu.make_async_copy(bufs[b], acc.at[didx.at[j]], sems_s[b]).wait()

    def block(blk, carry):
        base = wid * CPT + blk * BCH
        pltpu.sync_copy(src_hbm.at[pl.ds(base, BCH)], sidx)
        pltpu.sync_copy(dst_hbm.at[pl.ds(base, BCH)], didx)
        for b in range(nb):
            pltpu.async_copy(tbl_hbm.at[sidx.at[b]], bufs[b], sems_g[b])

        def step(q, c2):
            j = q * nb
            for b in range(nb):
                wait_g(b, j + b)
                pltpu.async_copy(bufs[b], acc.at[didx.at[j + b]],
                                 sems_s[b], add=True)
                if b >= 2:
                    b2 = b - 2
                    wait_s(b2, j + b2)
                    pltpu.async_copy(tbl_hbm.at[sidx.at[j + b2 + nb]],
                                     bufs[b2], sems_g[b2])
            for b in range(nb - 2, nb):
                wait_s(b, j + b)
                pltpu.async_copy(tbl_hbm.at[sidx.at[j + b + nb]],
                                 bufs[b], sems_g[b])
            return c2

        lax.fori_loop(0, BCH // nb - 1, step, 0)
        j0 = BCH - nb
        for b in range(nb):
            wait_g(b, j0 + b)
            pltpu.async_copy(bufs[b], acc.at[didx.at[j0 + b]],
                             sems_s[b], add=True)
        for b in range(nb):
            wait_s(b, j0 + b)
        return carry

    lax.fori_loop(0, CPT // BCH, block, 0)


def _make_agg_kernel(d, n_tables):
    mesh = plsc.VectorSubcoreMesh(core_axis_name="c", subcore_axis_name="s")
    out_type = jax.ShapeDtypeStruct((n_tables * 2 * NPAD, d), jnp.float32)
    scratch = (
        [pltpu.VMEM((BCH, 128), jnp.int32),
         pltpu.VMEM((BCH, 128), jnp.int32)]
        + [pltpu.VMEM((128, d), jnp.float32) for _ in range(NBUF)]
        + [pltpu.VMEM_SHARED((NPAD, d), jnp.float32)]
        + [pltpu.SemaphoreType.DMA for _ in range(2 * NBUF)]
    )

    def body(*refs):
        tbls = refs[:n_tables]
        src_hbm, dst_hbm, out_hbm = refs[n_tables:n_tables + 3]
        rest = refs[n_tables + 3:]
        sidx, didx = rest[0], rest[1]
        bufs = rest[2:2 + NBUF]
        acc = rest[2 + NBUF]
        sems_g = rest[3 + NBUF:3 + 2 * NBUF]
        sems_s = rest[3 + 2 * NBUF:3 + 3 * NBUF]
        c = lax.axis_index("c")
        s = lax.axis_index("s")
        wid = c * 16 + s
        row0 = s * ROWS_PER_TILE

        for p in range(n_tables):
            _zero_fill_buf(bufs[0], d)
            _zero_acc_rows(bufs[0], acc, row0)
            plsc.subcore_barrier()
            _edge_pass(tbls[p], src_hbm, dst_hbm, wid,
                       sidx, didx, bufs, sems_g, sems_s, acc)
            plsc.subcore_barrier()
            out0 = (p * 2 + c) * NPAD + row0
            pltpu.sync_copy(acc.at[pl.ds(row0, ROWS_PER_TILE)],
                            out_hbm.at[pl.ds(out0, ROWS_PER_TILE)])
            if p + 1 < n_tables:
                plsc.subcore_barrier()

    return pl.kernel(
        body, out_type=out_type, mesh=mesh, scratch_types=scratch,
        compiler_params=pltpu.CompilerParams(use_tc_tiling_on_sc=False))


_agg1 = _make_agg_kernel(16, 1)
_agg2 = _make_agg_kernel(32, 2)


NB = 8
RB = NPAD // NB


def _sortable_key(score):
    bits = lax.bitcast_convert_type(score, jnp.int32)
    return jnp.where(bits >= 0, bits, INT_MIN - bits)


def _conv1_dense_body(agg_a_ref, agg_b_ref, xp_ref, wrel_ref, brel_ref,
                      wroot_ref, p1_ref, h_ref, score_ref):
    agg = agg_a_ref[...] + agg_b_ref[...]
    h = jnp.dot(agg, wrel_ref[...], preferred_element_type=jnp.float32)
    h += jnp.dot(xp_ref[...], wroot_ref[...], preferred_element_type=jnp.float32)
    h = jnp.maximum(h + brel_ref[...], 0.0)
    h_ref[...] = h
    p1 = p1_ref[...]
    pnorm = jnp.sqrt(jnp.sum(p1 * p1))
    sc = lax.dot_general(h, p1, (((1,), (1,)), ((), ())),
                         preferred_element_type=jnp.float32)
    score_ref[...] = jnp.tanh(sc / pnorm)


_conv1_dense = pl.pallas_call(
    _conv1_dense_body,
    grid=(NB,),
    in_specs=[
        pl.BlockSpec((RB, 16), lambda i: (i, 0)),
        pl.BlockSpec((RB, 16), lambda i: (NB + i, 0)),
        pl.BlockSpec((RB, 16), lambda i: (i, 0)),
        pl.BlockSpec((16, 64), lambda i: (0, 0)),
        pl.BlockSpec((1, 64), lambda i: (0, 0)),
        pl.BlockSpec((16, 64), lambda i: (0, 0)),
        pl.BlockSpec((1, 64), lambda i: (0, 0)),
    ],
    out_specs=(
        pl.BlockSpec((RB, 64), lambda i: (i, 0)),
        pl.BlockSpec((RB, 1), lambda i: (i, 0)),
    ),
    out_shape=(
        jax.ShapeDtypeStruct((NPAD, 64), jnp.float32),
        jax.ShapeDtypeStruct((NPAD, 1), jnp.float32),
    ),
)


def _bisect_threshold(k, score2d, valid):
    skey = jnp.where(valid, _sortable_key(score2d), INT_MIN)
    kf = jnp.float32(k)

    def count_ge(t):
        return jnp.sum(jnp.where(skey >= t, 1.0, 0.0))

    t0 = jnp.where(count_ge(jnp.int32(0)) >= kf, jnp.int32(0), INT_MIN)

    def step(i, t):
        cand = t + lax.shift_left(jnp.int32(1), jnp.int32(30) - i)
        return jnp.where(count_ge(cand) >= kf, cand, t)

    return lax.fori_loop(0, 31, step, t0)


def _pool1_body(h_ref, score_ref, score2d_ref,
                g0_ref, g1_ref, alive_ref, x1p_ref, thr_s):
    i = pl.program_id(0)

    @pl.when(i == 0)
    def _():
        s2d = score2d_ref[...]
        idx = (lax.broadcasted_iota(jnp.int32, s2d.shape, 0) * 128
               + lax.broadcasted_iota(jnp.int32, s2d.shape, 1))
        thr_s[0] = _bisect_threshold(K1, s2d, idx < N)

    h = h_ref[...]
    score = score_ref[...]
    rows = i * RB + lax.broadcasted_iota(jnp.int32, (RB, 1), 0)
    valid = rows < N
    skey = jnp.where(valid, _sortable_key(score), INT_MIN)
    alive = skey >= thr_s[0]
    alive_f = jnp.where(alive, 1.0, 0.0)
    g = h * score * alive_f
    g0_ref[...] = g[:, 0:32]
    g1_ref[...] = g[:, 32:64]
    alive_ref[...] = alive_f
    prod = h * score
    bmax = jnp.max(jnp.where(alive, prod, -1e30), axis=0, keepdims=True)
    bsum = jnp.sum(g, axis=0, keepdims=True)
    blk = jnp.concatenate([bmax, bsum], axis=1)

    @pl.when(i == 0)
    def _():
        x1p_ref[...] = blk

    @pl.when(i > 0)
    def _():
        prev = x1p_ref[...]
        x1p_ref[...] = jnp.concatenate(
            [jnp.maximum(prev[:, 0:64], blk[:, 0:64]),
             prev[:, 64:128] + blk[:, 64:128]], axis=1)


_pool1 = pl.pallas_call(
    _pool1_body,
    grid=(NB,),
    in_specs=[
        pl.BlockSpec((RB, 64), lambda i: (i, 0)),
        pl.BlockSpec((RB, 1), lambda i: (i, 0)),
        pl.BlockSpec((NPAD // 128, 128), lambda i: (0, 0)),
    ],
    scratch_shapes=[pltpu.SMEM((1,), jnp.int32)],
    out_specs=(
        pl.BlockSpec((RB, 32), lambda i: (i, 0)),
        pl.BlockSpec((RB, 32), lambda i: (i, 0)),
        pl.BlockSpec((RB, 1), lambda i: (i, 0)),
        pl.BlockSpec((1, 128), lambda i: (0, 0)),
    ),
    out_shape=(
        jax.ShapeDtypeStruct((NPAD, 32), jnp.float32),
        jax.ShapeDtypeStruct((NPAD, 32), jnp.float32),
        jax.ShapeDtypeStruct((NPAD, 1), jnp.float32),
        jax.ShapeDtypeStruct((1, 128), jnp.float32),
    ),
)


def _conv2_dense_body(agg_a_ref, agg_b_ref, agg_c_ref, agg_d_ref,
                      g0_ref, g1_ref, wrel_ref, brel_ref, wroot_ref, p2_ref,
                      h_ref, score_ref):
    agg_lo = agg_a_ref[...] + agg_b_ref[...]
    agg_hi = agg_c_ref[...] + agg_d_ref[...]
    h = jnp.dot(agg_lo, wrel_ref[0:32, :], preferred_element_type=jnp.float32)
    h += jnp.dot(agg_hi, wrel_ref[32:64, :], preferred_element_type=jnp.float32)
    h += jnp.dot(g0_ref[...], wroot_ref[0:32, :], preferred_element_type=jnp.float32)
    h += jnp.dot(g1_ref[...], wroot_ref[32:64, :], preferred_element_type=jnp.float32)
    h = jnp.maximum(h + brel_ref[...], 0.0)
    h_ref[...] = h
    p2 = p2_ref[...]
    pnorm = jnp.sqrt(jnp.sum(p2 * p2))
    sc = lax.dot_general(h, p2, (((1,), (1,)), ((), ())),
                         preferred_element_type=jnp.float32)
    score_ref[...] = jnp.tanh(sc / pnorm)


_conv2_dense = pl.pallas_call(
    _conv2_dense_body,
    grid=(NB,),
    in_specs=[
        pl.BlockSpec((RB, 32), lambda i: (i, 0)),
        pl.BlockSpec((RB, 32), lambda i: (NB + i, 0)),
        pl.BlockSpec((RB, 32), lambda i: (2 * NB + i, 0)),
        pl.BlockSpec((RB, 32), lambda i: (3 * NB + i, 0)),
        pl.BlockSpec((RB, 32), lambda i: (i, 0)),
        pl.BlockSpec((RB, 32), lambda i: (i, 0)),
        pl.BlockSpec((64, 64), lambda i: (0, 0)),
        pl.BlockSpec((1, 64), lambda i: (0, 0)),
        pl.BlockSpec((64, 64), lambda i: (0, 0)),
        pl.BlockSpec((1, 64), lambda i: (0, 0)),
    ],
    out_specs=(
        pl.BlockSpec((RB, 64), lambda i: (i, 0)),
        pl.BlockSpec((RB, 1), lambda i: (i, 0)),
    ),
    out_shape=(
        jax.ShapeDtypeStruct((NPAD, 64), jnp.float32),
        jax.ShapeDtypeStruct((NPAD, 1), jnp.float32),
    ),
)


def _pool2_body(h_ref, score_ref, alive_ref, score2d_ref, alive2d_ref,
                x1p_ref, a1w_ref, a1b_ref, a5w_ref, a5b_ref,
                out_ref, thr_s, x2p_s):
    i = pl.program_id(0)

    @pl.when(i == 0)
    def _():
        thr_s[0] = _bisect_threshold(K2, score2d_ref[...],
                                     alive2d_ref[...] > 0.5)

    h = h_ref[...]
    score = score_ref[...]
    alive1 = alive_ref[...] > 0.5
    skey = jnp.where(alive1, _sortable_key(score), INT_MIN)
    alive2 = alive1 & (skey >= thr_s[0])
    prod = h * score
    bmax = jnp.max(jnp.where(alive2, prod, -1e30), axis=0, keepdims=True)
    bsum = jnp.sum(jnp.where(alive2, prod, 0.0), axis=0, keepdims=True)
    blk = jnp.concatenate([bmax, bsum], axis=1)

    @pl.when(i == 0)
    def _():
        x2p_s[...] = blk

    @pl.when(i > 0)
    def _():
        prev = x2p_s[...]
        x2p_s[...] = jnp.concatenate(
            [jnp.maximum(prev[:, 0:64], blk[:, 0:64]),
             prev[:, 64:128] + blk[:, 64:128]], axis=1)

    @pl.when(i == NB - 1)
    def _():
        x1p = x1p_ref[...]
        x2p = x2p_s[...]
        x1 = jnp.concatenate(
            [x1p[:, 0:64], x1p[:, 64:128] / jnp.float32(K1)], axis=1)
        x2 = jnp.concatenate(
            [x2p[:, 0:64], x2p[:, 64:128] / jnp.float32(K2)], axis=1)
        xp = x1 + x2
        a1 = jnp.maximum(
            jnp.dot(xp, a1w_ref[...], preferred_element_type=jnp.float32)
            + a1b_ref[...], 0.0)
        a5 = jnp.dot(a1, a5w_ref[...], preferred_element_type=jnp.float32) \
            + a5b_ref[...]
        out_ref[...] = jax.nn.softmax(a5, axis=0)


_pool2 = pl.pallas_call(
    _pool2_body,
    grid=(NB,),
    in_specs=[
        pl.BlockSpec((RB, 64), lambda i: (i, 0)),
        pl.BlockSpec((RB, 1), lambda i: (i, 0)),
        pl.BlockSpec((RB, 1), lambda i: (i, 0)),
        pl.BlockSpec((NPAD // 128, 128), lambda i: (0, 0)),
        pl.BlockSpec((NPAD // 128, 128), lambda i: (0, 0)),
        pl.BlockSpec((1, 128), lambda i: (0, 0)),
        pl.BlockSpec((128, 64), lambda i: (0, 0)),
        pl.BlockSpec((1, 64), lambda i: (0, 0)),
        pl.BlockSpec((64, 10), lambda i: (0, 0)),
        pl.BlockSpec((1, 10), lambda i: (0, 0)),
    ],
    out_specs=pl.BlockSpec((1, 10), lambda i: (0, 0)),
    out_shape=jax.ShapeDtypeStruct((1, 10), jnp.float32),
    scratch_shapes=[pltpu.SMEM((1,), jnp.int32),
                    pltpu.VMEM((1, 128), jnp.float32)],
)


@jax.jit
def kernel(x, edge_index, batch, W_rel1, b_rel1, W_root1, p1,
           W_rel2, b_rel2, W_root2, p2, A1, bA1, A5, bA5):
    src = edge_index[0]
    dst = edge_index[1]
    pad = jnp.full((EPAD - E,), N, jnp.int32)
    srcc = jnp.concatenate([src, pad]).reshape(ECH, 128)
    dstc = jnp.concatenate([dst, pad]).reshape(ECH, 128)

    x_pad = jnp.zeros((NPAD, 16), jnp.float32).at[:N, :12].set(x)
    wrel1 = jnp.zeros((16, 64), jnp.float32).at[:12, :].set(W_rel1)
    wroot1 = jnp.zeros((16, 64), jnp.float32).at[:12, :].set(W_root1)

    aggp1 = _agg1(x_pad, srcc, dstc)
    h1, score1 = _conv1_dense(
        aggp1, aggp1, x_pad, wrel1, b_rel1.reshape(1, 64), wroot1,
        p1.reshape(1, 64))
    g0, g1, alive, x1p = _pool1(h1, score1, score1.reshape(NPAD // 128, 128))

    aggp2 = _agg2(g0, g1, srcc, dstc)
    h2, score2 = _conv2_dense(
        aggp2, aggp2, aggp2, aggp2, g0, g1,
        W_rel2, b_rel2.reshape(1, 64), W_root2, p2.reshape(1, 64))
    out = _pool2(h2, score2, alive,
                 score2.reshape(NPAD // 128, 128),
                 alive.reshape(NPAD // 128, 128),
                 x1p, A1, bA1.reshape(1, 64), A5, bA5.reshape(1, 10))
    return out[None]
